# baseline (speedup 1.0000x reference)
import functools
import math

import jax
import jax.numpy as jnp
from jax import lax
from jax.experimental import pallas as pl
from jax.experimental.pallas import tpu as pltpu
from jax.experimental.pallas import tpu_sc as plsc

NUM_USERS = 50000
NUM_ITEMS = 50000
N_TOTAL = NUM_USERS + NUM_ITEMS
LATENT_DIM = 32
N_LAYERS = 3
BATCH = 4096
N_EDGES = 1600000
HALF_EDGES = N_EDGES // 2
TIME_BINS = 24
GAUSS_SIGMA = 2.0
CLOCK_ALPHA = 0.5

PADH = 48
NUP = NUM_USERS + PADH
NTP = 2 * NUP

NC = 2
NS = 16
NW = NC * NS
LANES = 16

EPW = 51200
EPADH = EPW * NS
EPAD = EPADH - HALF_EDGES
SUP = 3200
NSUP = EPW // SUP
GPSUP = SUP // 128
CHUNK = 640
GPC = CHUNK // 128
CPS = SUP // CHUNK
ROWS_PT = NUP // NS
BPW = BATCH // NW

_mesh = plsc.VectorSubcoreMesh(core_axis_name="c", subcore_axis_name="s")
_f32 = jnp.float32
_i32 = jnp.int32


def _layer_body(xprev, cols2d, rows2d, vals1d, xnext,
                acc, idxc, idxr, valv, gv, sem):
    c = lax.axis_index("c")
    s = lax.axis_index("s")
    zero = jnp.zeros((LANES,), _f32)

    def _z(i, _):
        gv[i, pl.ds(0, 16)] = zero
        gv[i, pl.ds(16, 16)] = zero
        return 0
    lax.fori_loop(0, CHUNK, _z, 0)
    rbase = s * ROWS_PT
    subchunks = []
    o = 0
    while o < ROWS_PT:
        n = min(CHUNK, ROWS_PT - o)
        subchunks.append((o, n))
        o += n
    for o, n in subchunks:
        pltpu.sync_copy(gv.at[pl.ds(0, n)], acc.at[pl.ds(rbase + o, n)])
    plsc.subcore_barrier()

    gbase = c * (EPADH // 128) + s * (EPW // 128)
    ebase = c * EPADH + s * EPW
    rowoff = c * NUP

    def _super(ks, _):
        goff = gbase + ks * GPSUP
        pltpu.sync_copy(cols2d.at[pl.ds(goff, GPSUP)], idxc)
        pltpu.sync_copy(rows2d.at[pl.ds(goff, GPSUP)], idxr)
        pltpu.sync_copy(vals1d.at[pl.ds(ebase + ks * SUP, SUP)], valv)

        def _loc(j, _):
            for t in range(8):
                idxr[j, pl.ds(t * 16, 16)] = (
                    idxr[j, pl.ds(t * 16, 16)] - rowoff)
            return 0
        lax.fori_loop(0, GPSUP, _loc, 0)

        for cq in range(CPS):
            descs = []
            for j in range(GPC):
                descs.append(pltpu.async_copy(
                    xprev.at[idxc.at[cq * GPC + j]],
                    gv.at[pl.ds(j * 128, 128)], sem))
            for d in descs:
                d.wait()

            def _sc(g, _):
                vv = valv[pl.ds(cq * CHUNK + g * 16, 16)]
                for l in range(16):
                    v = vv[l]
                    e = g * 16 + l
                    gv[e, pl.ds(0, 16)] = gv[e, pl.ds(0, 16)] * v
                    gv[e, pl.ds(16, 16)] = gv[e, pl.ds(16, 16)] * v
                return 0
            lax.fori_loop(0, CHUNK // 16, _sc, 0)

            for j in range(GPC):
                pltpu.sync_copy(gv.at[pl.ds(j * 128, 128)],
                                acc.at[idxr.at[cq * GPC + j]], add=True)
        return 0
    lax.fori_loop(0, NSUP, _super, 0)
    plsc.subcore_barrier()

    obase = c * NUP + rbase
    for o, n in subchunks:
        pltpu.sync_copy(acc.at[pl.ds(rbase + o, n)], gv.at[pl.ds(0, n)])
        pltpu.sync_copy(gv.at[pl.ds(0, n)], xnext.at[pl.ds(obase + o, n)])


_sc_params = pltpu.CompilerParams(use_tc_tiling_on_sc=False,
                                  needs_layout_passes=False)

_layer = functools.partial(
    pl.kernel,
    out_type=jax.ShapeDtypeStruct((NTP, LATENT_DIM), _f32),
    mesh=_mesh,
    compiler_params=_sc_params,
    scratch_types=[
        pltpu.VMEM_SHARED((NUP, LATENT_DIM), _f32),
        pltpu.VMEM((GPSUP, 128), _i32),
        pltpu.VMEM((GPSUP, 128), _i32),
        pltpu.VMEM((SUP,), _f32),
        pltpu.VMEM((CHUNK, LATENT_DIM), _f32),
        pltpu.SemaphoreType.DMA,
    ],
)(_layer_body)


def _batch_body(users, pos, neg, thetas, x0, x1, x2, x3, cat, top3r, icats2,
                ps, ns, regp,
                ub, pbr, nbr, pba, nba, thv, tmp, tmp2, usum, psum, nsum,
                t3, icp, icn, tflat, hcall, outp, outn, regv, ctb,
                sem, semb, semc):
    c = lax.axis_index("c")
    s = lax.axis_index("s")
    w = c * NS + s
    b0 = w * BPW
    zero = jnp.zeros((LANES,), _f32)

    pltpu.sync_copy(users.at[pl.ds(b0, BPW)], ub)
    pltpu.sync_copy(pos.at[pl.ds(b0, BPW)], pbr)
    pltpu.sync_copy(neg.at[pl.ds(b0, BPW)], nbr)
    pltpu.sync_copy(thetas.at[pl.ds(b0, BPW)], thv)

    d_t3 = pltpu.async_copy(top3r.at[ub], t3, semc)
    d_icp = pltpu.async_copy(icats2.at[pbr], icp, semc)
    d_icn = pltpu.async_copy(icats2.at[nbr], icn, semc)

    def _adj(i, _):
        pba[pl.ds(i * 16, 16)] = pbr[pl.ds(i * 16, 16)] + NUP
        nba[pl.ds(i * 16, 16)] = nbr[pl.ds(i * 16, 16)] + NUP
        return 0
    lax.fori_loop(0, BPW // 16, _adj, 0)

    regv[pl.ds(0, 16)] = zero

    def _zs(i, _):
        for h in (0, 16):
            usum[i, pl.ds(h, 16)] = zero
            psum[i, pl.ds(h, 16)] = zero
            nsum[i, pl.ds(h, 16)] = zero
        return 0
    lax.fori_loop(0, BPW, _zs, 0)

    seq = [(x0, ub, usum, True), (x1, ub, usum, False),
           (x2, ub, usum, False), (x3, ub, usum, False),
           (x0, pba, psum, True), (x1, pba, psum, False),
           (x2, pba, psum, False), (x3, pba, psum, False),
           (x0, nba, nsum, True), (x1, nba, nsum, False),
           (x2, nba, nsum, False), (x3, nba, nsum, False)]
    bufs = [tmp, tmp2]
    sems = [sem, semb]
    descs = [None, None]
    descs[0] = pltpu.async_copy(seq[0][0].at[seq[0][1]], bufs[0], sems[0])
    for i in range(len(seq)):
        if i + 1 < len(seq):
            xk, idxref, _, _ = seq[i + 1]
            descs[(i + 1) % 2] = pltpu.async_copy(
                xk.at[idxref], bufs[(i + 1) % 2], sems[(i + 1) % 2])
        descs[i % 2].wait()
        buf = bufs[i % 2]
        accum = seq[i][2]
        with_sq = seq[i][3]

        def _a(r, _):
            for h in (0, 16):
                t = buf[r, pl.ds(h, 16)]
                accum[r, pl.ds(h, 16)] = accum[r, pl.ds(h, 16)] + t
                if with_sq:
                    regv[pl.ds(0, 16)] = regv[pl.ds(0, 16)] + t * t
            return 0
        lax.fori_loop(0, BPW, _a, 0)

    d_t3.wait()
    d_icp.wait()
    d_icn.wait()

    ioti = lax.iota(_i32, 16)

    def _bt(e, _):
        base = e * 80
        for t in range(4):
            tflat[pl.ds(base + t * 16, 16)] = t3[e, pl.ds(t * 16, 16)]
        tail = t3[e, pl.ds(64, 16)]
        icpv = icp[e, pl.ds(0, 16)]
        icnv = icn[e, pl.ds(0, 16)]
        tail = jnp.where(ioti == 8, icpv[0], tail)
        tail = jnp.where(ioti == 9, icnv[0], tail)
        tflat[pl.ds(base + 64, 16)] = tail
        return 0
    lax.fori_loop(0, BPW, _bt, 0)

    inv2pi24 = TIME_BINS / (2.0 * math.pi)
    neg_half_inv_sig2 = -1.0 / (2.0 * GAUSS_SIGMA * GAUSS_SIGMA)
    iot = lax.iota(_i32, 16).astype(_f32)
    hbl = iot
    hbh = iot + 16.0
    maskh = hbh < float(TIME_BINS)

    def _blk(g, _):
        e0 = g * 16
        descs = []
        for j in range(10):
            descs.append(pltpu.async_copy(
                cat.at[tflat.at[pl.ds(e0 * 80 + j * 128, 128)]],
                hcall.at[pl.ds(j * 128, 128)], sem))
        for d in descs:
            d.wait()
        tvec = thv[pl.ds(e0, 16)]
        pvec = []
        nvec = []
        for l in range(16):
            e = e0 + l
            rb = l * 80
            th = tvec[l]
            cur = th * inv2pi24
            dl = jnp.abs(cur - hbl)
            dl = jnp.minimum(dl, 24.0 - dl)
            dh = jnp.abs(cur - hbh)
            dh = jnp.minimum(dh, 24.0 - dh)
            wl = jnp.exp(dl * dl * neg_half_inv_sig2)
            wh = jnp.exp(dh * dh * neg_half_inv_sig2)
            wh = jnp.where(maskh, wh, 0.0)
            sumw = jnp.sum(wl) + jnp.sum(wh) + 1e-08
            scale_vec = jnp.full((16,), 1.0 / 3.0, _f32) / (
                jnp.zeros((16,), _f32) + sumw)
            wln = wl * scale_vec
            whn = wh * scale_vec

            v0 = zero
            v1 = zero
            for h in range(TIME_BINS):
                cf = wln[h] if h < 16 else whn[h - 16]
                for r in range(3):
                    j = rb + 3 * h + r
                    v0 = v0 + hcall[j, pl.ds(0, 16)] * cf
                    v1 = v1 + hcall[j, pl.ds(16, 16)] * cf
            clock_pos = (jnp.sum(v0 * hcall[rb + 72, pl.ds(0, 16)])
                         + jnp.sum(v1 * hcall[rb + 72, pl.ds(16, 16)]))
            clock_neg = (jnp.sum(v0 * hcall[rb + 73, pl.ds(0, 16)])
                         + jnp.sum(v1 * hcall[rb + 73, pl.ds(16, 16)]))

            u0 = usum[e, pl.ds(0, 16)]
            u1 = usum[e, pl.ds(16, 16)]
            p0 = psum[e, pl.ds(0, 16)]
            p1 = psum[e, pl.ds(16, 16)]
            n0 = nsum[e, pl.ds(0, 16)]
            n1 = nsum[e, pl.ds(16, 16)]
            base_pos = (jnp.sum(u0 * p0) + jnp.sum(u1 * p1)) * (1.0 / 16.0)
            base_neg = (jnp.sum(u0 * n0) + jnp.sum(u1 * n1)) * (1.0 / 16.0)
            pvec.append(base_pos + CLOCK_ALPHA * clock_pos)
            nvec.append(base_neg + CLOCK_ALPHA * clock_neg)
        pv = zero
        nv = zero
        for l in range(16):
            lane = ioti == l
            pv = jnp.where(lane, pvec[l], pv)
            nv = jnp.where(lane, nvec[l], nv)
        outp[pl.ds(e0, 16)] = pv
        outn[pl.ds(e0, 16)] = nv
        return 0
    lax.fori_loop(0, BPW // 16, _blk, 0)

    @pl.when(w == 0)
    def _cat_reg():
        def _cc(i, _):
            pltpu.sync_copy(cat.at[pl.ds(i * 200, 200)], ctb)

            def _sq(r, _):
                a = ctb[r, pl.ds(0, 16)]
                b = ctb[r, pl.ds(16, 16)]
                regv[pl.ds(0, 16)] = regv[pl.ds(0, 16)] + a * a + b * b
                return 0
            lax.fori_loop(0, 200, _sq, 0)
            return 0
        lax.fori_loop(0, 5, _cc, 0)

    pltpu.sync_copy(outp, ps.at[pl.ds(b0, BPW)])
    pltpu.sync_copy(outn, ns.at[pl.ds(b0, BPW)])
    pltpu.sync_copy(regv, regp.at[pl.ds(w * LANES, LANES)])


_batch = functools.partial(
    pl.kernel,
    out_type=(
        jax.ShapeDtypeStruct((BATCH,), _f32),
        jax.ShapeDtypeStruct((BATCH,), _f32),
        jax.ShapeDtypeStruct((NW * LANES,), _f32),
    ),
    mesh=_mesh,
    compiler_params=_sc_params,
    scratch_types=[
        pltpu.VMEM((BPW,), _i32),
        pltpu.VMEM((BPW,), _i32),
        pltpu.VMEM((BPW,), _i32),
        pltpu.VMEM((BPW,), _i32),
        pltpu.VMEM((BPW,), _i32),
        pltpu.VMEM((BPW,), _f32),
        pltpu.VMEM((BPW, LATENT_DIM), _f32),
        pltpu.VMEM((BPW, LATENT_DIM), _f32),
        pltpu.VMEM((BPW, LATENT_DIM), _f32),
        pltpu.VMEM((BPW, LATENT_DIM), _f32),
        pltpu.VMEM((BPW, LATENT_DIM), _f32),
        pltpu.VMEM((BPW, 80), _i32),
        pltpu.VMEM((BPW, 16), _i32),
        pltpu.VMEM((BPW, 16), _i32),
        pltpu.VMEM((BPW * 80,), _i32),
        pltpu.VMEM((1280, LATENT_DIM), _f32),
        pltpu.VMEM((BPW,), _f32),
        pltpu.VMEM((BPW,), _f32),
        pltpu.VMEM((LANES,), _f32),
        pltpu.VMEM((200, LATENT_DIM), _f32),
        pltpu.SemaphoreType.DMA,
        pltpu.SemaphoreType.DMA,
        pltpu.SemaphoreType.DMA,
    ],
)(_batch_body)


def _final_tc(psr, nsr, regpr, bpr_ref, reg_ref):
    x = nsr[...] - psr[...]
    sp = jnp.log1p(jnp.exp(-jnp.abs(x))) + jnp.maximum(x, 0.0)
    bpr_ref[...] = jnp.reshape(jnp.sum(sp) * (1.0 / BATCH), (1, 1))
    reg_ref[...] = jnp.reshape(jnp.sum(regpr[...]) * (0.5 / BATCH), (1, 1))


def kernel(users, pos, neg, thetas, user_table, item_table, cat_table,
           rows, cols, vals, item_cats, top3):
    users = users.astype(_i32)
    pos = pos.astype(_i32)
    neg = neg.astype(_i32)
    rows = rows.astype(_i32)
    cols = cols.astype(_i32)

    zrow = jnp.zeros((PADH, LATENT_DIM), _f32)
    x0 = jnp.concatenate([user_table, zrow, item_table, zrow], axis=0)

    zi = jnp.zeros((EPAD,), _i32)
    zf = jnp.zeros((EPAD,), _f32)
    cols_p = jnp.concatenate([cols[:HALF_EDGES] + PADH, zi,
                              cols[HALF_EDGES:], zi])
    rows_p = jnp.concatenate([rows[:HALF_EDGES], zi,
                              rows[HALF_EDGES:] + PADH,
                              jnp.full((EPAD,), NUP, _i32)])
    vals_p = jnp.concatenate([vals[:HALF_EDGES], zf, vals[HALF_EDGES:], zf])
    cols2d = cols_p.reshape(-1, 128)
    rows2d = rows_p.reshape(-1, 128)

    x1 = _layer(x0, cols2d, rows2d, vals_p)
    x2 = _layer(x1, cols2d, rows2d, vals_p)
    x3 = _layer(x2, cols2d, rows2d, vals_p)

    top3r = jnp.pad(top3.reshape(NUM_USERS, TIME_BINS * 3).astype(_i32),
                    ((0, 0), (0, 8)))
    icats2 = jnp.pad(item_cats.astype(_i32).reshape(NUM_ITEMS, 1),
                     ((0, 0), (0, 15)))

    ps, nsc, regp = _batch(users, pos, neg, thetas, x0, x1, x2, x3,
                           cat_table, top3r, icats2)

    bpr, reg = pl.pallas_call(
        _final_tc,
        out_shape=(jax.ShapeDtypeStruct((1, 1), _f32),
                   jax.ShapeDtypeStruct((1, 1), _f32)),
    )(ps.reshape(32, 128), nsc.reshape(32, 128), regp.reshape(4, 128))

    return (bpr.reshape(()), reg.reshape(()), jnp.zeros(()))
